# Initial kernel scaffold; baseline (speedup 1.0000x reference)
import jax
import jax.numpy as jnp
from jax.experimental import pallas as pl


def kernel(atoms, pos, edge_index, batch, emb_table, proj_W, proj_b, msg_W1, msg_b1, msg_g1, msg_be1, msg_W2, msg_b2, msg_g2, msg_be2, pos_W1, pos_b1, pos_g1, pos_be1, pos_W2, pos_b2, upd_W1, upd_b1, upd_g1, upd_be1, upd_W2, upd_b2, upd_g2, upd_be2, pred_W1, pred_b1, pred_W2, pred_b2):
    raise NotImplementedError("write your pallas kernel here")


# speedup vs baseline: 3.6562x; 3.6562x over previous
import functools

import jax
import jax.numpy as jnp
from jax import lax
from jax.experimental import pallas as pl
from jax.experimental.pallas import tpu as pltpu
from jax.experimental.pallas import tpu_sc as plsc

N = 10000
E = 320000
EMB = 128
VOCAB = 100
NLAYERS = 5
G = 64
PP = 16
TW = 2 * EMB
NC, NS = 2, 16
NW = NC * NS
CH = 128
NCH = E // CH
NPAD = 10240
ROWS_PER_SUB = NPAD // NS
CHP = 80
NCHP = N // CHP
BE = 1000
BN = 1000
F32 = jnp.float32

_mesh = plsc.VectorSubcoreMesh(core_axis_name="c", subcore_axis_name="s",
                               num_cores=NC, num_subcores=NS)
_SDS = jax.ShapeDtypeStruct


@functools.partial(
    pl.kernel,
    out_type=[_SDS((E, TW), F32), _SDS((E, TW), F32)],
    mesh=_mesh,
    scratch_types=[pltpu.VMEM((CH,), jnp.int32), pltpu.VMEM((CH,), jnp.int32),
                   pltpu.VMEM((CH, TW), F32), pltpu.VMEM((CH, TW), F32),
                   pltpu.SemaphoreType.DMA],
)
def _sc_gather(t1_h, t2_h, d_h, s_h, ag_h, bg_h, idxd, idxs, bufa, bufb, sem):
    w = lax.axis_index("s") * NC + lax.axis_index("c")
    nloop = (NCH + NW - 1) // NW

    @pl.loop(0, nloop)
    def _(j):
        c = j * NW + w

        @pl.when(c < NCH)
        def _():
            e0 = pl.multiple_of(c * CH, 8)
            pltpu.sync_copy(d_h.at[pl.ds(e0, CH)], idxd)
            pltpu.sync_copy(s_h.at[pl.ds(e0, CH)], idxs)
            cp1 = pltpu.async_copy(t1_h.at[idxd], bufa, sem)
            cp2 = pltpu.async_copy(t2_h.at[idxs], bufb, sem)
            cp1.wait()
            cp2.wait()
            pltpu.sync_copy(bufa, ag_h.at[pl.ds(e0, CH)])
            pltpu.sync_copy(bufb, bg_h.at[pl.ds(e0, CH)])


@functools.partial(
    pl.kernel,
    out_type=[_SDS((NC, NPAD, EMB), F32), _SDS((NC, NPAD, EMB), F32)],
    mesh=_mesh,
    scratch_types=[pltpu.VMEM((CH,), jnp.int32),
                   pltpu.VMEM((CH, EMB), F32),
                   pltpu.VMEM_SHARED((NPAD, EMB), F32)],
)
def _sc_scatter(m_h, pd_h, d_h, zm_h, om_h, op_h, idx, rows, acc):
    cid = lax.axis_index("c")
    sid = lax.axis_index("s")
    w = sid * NC + cid
    r0 = pl.multiple_of(sid * ROWS_PER_SUB, 8)
    nloop = (NCH + NW - 1) // NW

    for src_h, out_h in ((m_h, om_h), (pd_h, op_h)):
        pltpu.sync_copy(zm_h.at[pl.ds(r0, ROWS_PER_SUB)],
                        acc.at[pl.ds(r0, ROWS_PER_SUB)])
        plsc.subcore_barrier()

        @pl.loop(0, nloop)
        def _(j, src_h=src_h):
            c = j * NW + w

            @pl.when(c < NCH)
            def _():
                e0 = pl.multiple_of(c * CH, 8)
                pltpu.sync_copy(d_h.at[pl.ds(e0, CH)], idx)
                pltpu.sync_copy(src_h.at[pl.ds(e0, CH)], rows)
                pltpu.sync_copy(rows, acc.at[idx], add=True)

        plsc.subcore_barrier()
        pltpu.sync_copy(acc.at[pl.ds(r0, ROWS_PER_SUB)],
                        out_h.at[cid, pl.ds(r0, ROWS_PER_SUB)])
        plsc.subcore_barrier()


@functools.partial(
    pl.kernel,
    out_type=_SDS((NC, G, EMB), F32),
    mesh=_mesh,
    scratch_types=[pltpu.VMEM((CHP,), jnp.int32),
                   pltpu.VMEM((CHP, EMB), F32),
                   pltpu.VMEM_SHARED((G, EMB), F32)],
)
def _sc_pool(h_h, b_h, zm_h, og_h, idx, rows, acc):
    cid = lax.axis_index("c")
    sid = lax.axis_index("s")
    w = sid * NC + cid
    gper = 8
    r0 = pl.multiple_of(sid * gper, 8)

    @pl.when(sid < G // gper)
    def _():
        pltpu.sync_copy(zm_h.at[pl.ds(r0, gper)], acc.at[pl.ds(r0, gper)])

    plsc.subcore_barrier()
    nloop = (NCHP + NW - 1) // NW

    @pl.loop(0, nloop)
    def _(j):
        c = j * NW + w

        @pl.when(c < NCHP)
        def _():
            e0 = pl.multiple_of(c * CHP, 8)
            pltpu.sync_copy(b_h.at[pl.ds(e0, CHP)], idx)
            pltpu.sync_copy(h_h.at[pl.ds(e0, CHP)], rows)
            pltpu.sync_copy(rows, acc.at[idx], add=True)

    plsc.subcore_barrier()

    @pl.when(sid < G // gper)
    def _():
        pltpu.sync_copy(acc.at[pl.ds(r0, gper)], og_h.at[cid, pl.ds(r0, gper)])


def _ln(x, g, b):
    mu = jnp.mean(x, axis=-1, keepdims=True)
    var = jnp.mean((x - mu) * (x - mu), axis=-1, keepdims=True)
    return (x - mu) * jax.lax.rsqrt(var + 1e-5) * g + b


def _dot(a, b):
    return jnp.dot(a, b, preferred_element_type=F32)


def _vecspec():
    return pl.BlockSpec((1, EMB), lambda i: (0, 0))


def _matspec():
    return pl.BlockSpec((EMB, EMB), lambda i: (0, 0))


def _table(a, pos16):
    z = jnp.zeros((a.shape[0], TW - EMB - PP), F32)
    return jnp.concatenate([a, pos16, z], axis=1)


def _tc_embed(atoms, pos16, emb_pad, proj_w, proj_b, w1a, w1b):
    def body(at_ref, p_ref, et_ref, pw_ref, pb_ref, wa_ref, wb_ref,
             h_ref, t1_ref, t2_ref):
        a = at_ref[...]
        ioh = lax.broadcasted_iota(jnp.int32, (BN, EMB), 1)
        oh = (ioh == a).astype(F32)
        h = _dot(oh, et_ref[...])
        h = _dot(h, pw_ref[...]) + pb_ref[...]
        h_ref[...] = h
        p = p_ref[...]
        t1_ref[...] = _table(_dot(h, wa_ref[...]), p)
        t2_ref[...] = _table(_dot(h, wb_ref[...]), p)

    return pl.pallas_call(
        body,
        grid=(N // BN,),
        in_specs=[pl.BlockSpec((BN, 1), lambda i: (i, 0)),
                  pl.BlockSpec((BN, PP), lambda i: (i, 0)),
                  _matspec(), _matspec(), _vecspec(), _matspec(), _matspec()],
        out_specs=[pl.BlockSpec((BN, EMB), lambda i: (i, 0)),
                   pl.BlockSpec((BN, TW), lambda i: (i, 0)),
                   pl.BlockSpec((BN, TW), lambda i: (i, 0))],
        out_shape=[_SDS((N, EMB), F32), _SDS((N, TW), F32),
                   _SDS((N, TW), F32)],
    )(atoms, pos16, emb_pad, proj_w, proj_b, w1a, w1b)


def _tc_edge(first_layer, ag, bg, w1c, b1, g1, be1, w2, b2, g2, be2,
             pw1, pb1, pg1, pbe1, pw2r, pb2):
    def body(ag_ref, bg_ref, w1c_ref, b1_ref, g1_ref, be1_ref,
             w2_ref, b2_ref, g2_ref, be2_ref, pw1_ref, pb1_ref, pg1_ref,
             pbe1_ref, pw2r_ref, pb2_ref, m_ref, pdo_ref):
        t1 = ag_ref[...]
        t2 = bg_ref[...]
        diff = t1[:, EMB:] - t2[:, EMB:]
        d2 = jnp.sum(diff * diff, axis=-1, keepdims=True)
        dist = jnp.sqrt(d2 + 1e-12)
        pre = t1[:, :EMB] + t2[:, :EMB] + dist * w1c_ref[...] + b1_ref[...]
        m = jax.nn.relu(_ln(pre, g1_ref[...], be1_ref[...]))
        m = jax.nn.relu(_ln(_dot(m, w2_ref[...]) + b2_ref[...],
                            g2_ref[...], be2_ref[...]))
        pw = jax.nn.relu(_ln(_dot(m, pw1_ref[...]) + pb1_ref[...],
                             pg1_ref[...], pbe1_ref[...]))
        s = jnp.sum(pw * pw2r_ref[...], axis=-1, keepdims=True) + pb2_ref[...]
        pdv = diff * s
        if first_layer:
            lanes = lax.broadcasted_iota(jnp.int32, (BE, EMB), 1)
            pdv = jnp.where(lanes == 3, 1.0, pdv)
        m_ref[...] = m
        pdo_ref[...] = pdv

    espec = pl.BlockSpec((BE, EMB), lambda i: (i, 0))
    tspec = pl.BlockSpec((BE, TW), lambda i: (i, 0))
    return pl.pallas_call(
        body,
        grid=(E // BE,),
        in_specs=[tspec, tspec,
                  _vecspec(), _vecspec(), _vecspec(), _vecspec(),
                  _matspec(), _vecspec(), _vecspec(), _vecspec(),
                  _matspec(), _vecspec(), _vecspec(), _vecspec(),
                  _vecspec(), pl.BlockSpec((1, 1), lambda i: (0, 0))],
        out_specs=[espec, espec],
        out_shape=[_SDS((E, EMB), F32), _SDS((E, EMB), F32)],
    )(ag, bg, w1c, b1, g1, be1, w2, b2, g2, be2,
      pw1, pb1, pg1, pbe1, pw2r, pb2)


def _tc_node(is_first, has_next, h, mp, pp, pos16, deginv,
             u1a, u1b, ub1, ug1, ube1, u2, ub2, ug2, ube2, nw1a, nw1b):
    def body(*refs):
        (h_ref, mp_ref, pp_ref, pos_ref, di_ref, u1a_ref, u1b_ref, ub1_ref,
         ug1_ref, ube1_ref, u2_ref, ub2_ref, ug2_ref, ube2_ref,
         nwa_ref, nwb_ref) = refs[:16]
        orefs = refs[16:]
        h = h_ref[...]
        ma = mp_ref[0] + mp_ref[1]
        pa = pp_ref[0] + pp_ref[1]
        if is_first:
            cnt = pa[:, 3:4]
            di = 1.0 / jnp.maximum(cnt, 1.0)
        else:
            di = di_ref[...][:, 0:1]
        lanes = lax.broadcasted_iota(jnp.int32, (BN, PP), 1)
        pos_ag = jnp.where(lanes < 3, pa[:, :PP] * di, 0.0)
        u = jax.nn.relu(_ln(_dot(h, u1a_ref[...]) + _dot(ma, u1b_ref[...])
                            + ub1_ref[...], ug1_ref[...], ube1_ref[...]))
        u = jax.nn.relu(_ln(_dot(u, u2_ref[...]) + ub2_ref[...],
                            ug2_ref[...], ube2_ref[...]))
        hn = h + u
        k = 0
        orefs[k][...] = hn
        k += 1
        if has_next:
            posn = pos_ref[...] + pos_ag
            orefs[k][...] = posn
            k += 1
            orefs[k][...] = _table(_dot(hn, nwa_ref[...]), posn)
            k += 1
            orefs[k][...] = _table(_dot(hn, nwb_ref[...]), posn)
            k += 1
        if is_first:
            orefs[k][...] = jnp.broadcast_to(di, (BN, PP))
            k += 1

    nspec = pl.BlockSpec((BN, EMB), lambda i: (i, 0))
    pspec = pl.BlockSpec((BN, PP), lambda i: (i, 0))
    tspec = pl.BlockSpec((BN, TW), lambda i: (i, 0))
    out_specs = [nspec]
    out_shape = [_SDS((N, EMB), F32)]
    if has_next:
        out_specs += [pspec, tspec, tspec]
        out_shape += [_SDS((N, PP), F32), _SDS((N, TW), F32),
                      _SDS((N, TW), F32)]
    if is_first:
        out_specs += [pspec]
        out_shape += [_SDS((N, PP), F32)]
    return pl.pallas_call(
        body,
        grid=(N // BN,),
        in_specs=[nspec,
                  pl.BlockSpec((NC, BN, EMB), lambda i: (0, i, 0)),
                  pl.BlockSpec((NC, BN, EMB), lambda i: (0, i, 0)),
                  pspec, pspec,
                  _matspec(), _matspec(), _vecspec(), _vecspec(), _vecspec(),
                  _matspec(), _vecspec(), _vecspec(), _vecspec(),
                  _matspec(), _matspec()],
        out_specs=out_specs,
        out_shape=out_shape,
    )(h, mp, pp, pos16, deginv, u1a, u1b, ub1, ug1, ube1,
      u2, ub2, ug2, ube2, nw1a, nw1b)


def _tc_pred(gp, w1, b1, w2r, b2):
    def body(gp_ref, w1_ref, b1_ref, w2r_ref, b2_ref, o_ref):
        g = gp_ref[0] + gp_ref[1]
        r = jax.nn.relu(_dot(g, w1_ref[...]) + b1_ref[...])
        o_ref[...] = jnp.sum(r * w2r_ref[...], axis=-1, keepdims=True) \
            + b2_ref[...]

    return pl.pallas_call(
        body,
        grid=(1,),
        in_specs=[pl.BlockSpec((NC, G, EMB), lambda i: (0, 0, 0)),
                  _matspec(), _vecspec(), _vecspec(),
                  pl.BlockSpec((1, 1), lambda i: (0, 0))],
        out_specs=pl.BlockSpec((G, 1), lambda i: (0, 0)),
        out_shape=_SDS((G, 1), F32),
    )(gp, w1, b1, w2r, b2)


def kernel(atoms, pos, edge_index, batch, emb_table, proj_W, proj_b,
           msg_W1, msg_b1, msg_g1, msg_be1, msg_W2, msg_b2, msg_g2, msg_be2,
           pos_W1, pos_b1, pos_g1, pos_be1, pos_W2, pos_b2,
           upd_W1, upd_b1, upd_g1, upd_be1, upd_W2, upd_b2, upd_g2, upd_be2,
           pred_W1, pred_b1, pred_W2, pred_b2):
    f = lambda x: x.astype(F32)
    src = edge_index[0].astype(jnp.int32)
    dst = edge_index[1].astype(jnp.int32)
    batch_i = batch.astype(jnp.int32)
    pos16 = jnp.concatenate([f(pos), jnp.zeros((N, PP - 3), F32)], axis=1)
    emb_pad = jnp.pad(f(emb_table), ((0, EMB - VOCAB), (0, 0)))
    zmsg = jnp.zeros((NPAD, EMB), F32)
    row = lambda v: v.reshape(1, -1).astype(F32)
    one = lambda v: v.reshape(1, 1).astype(F32)

    h, t1, t2 = _tc_embed(atoms.astype(jnp.int32), pos16, emb_pad, f(proj_W),
                          row(proj_b), msg_W1[0, :EMB, :],
                          msg_W1[0, EMB:2 * EMB, :])
    deginv = pos16
    for i in range(NLAYERS):
        ag, bg = _sc_gather(t1, t2, dst, src)
        m, pd = _tc_edge(i == 0, ag, bg,
                         row(msg_W1[i, 2 * EMB, :]), row(msg_b1[i]),
                         row(msg_g1[i]), row(msg_be1[i]),
                         f(msg_W2[i]), row(msg_b2[i]), row(msg_g2[i]),
                         row(msg_be2[i]),
                         f(pos_W1[i]), row(pos_b1[i]), row(pos_g1[i]),
                         row(pos_be1[i]),
                         row(pos_W2[i][:, 0]), one(pos_b2[i]))
        mp, pp = _sc_scatter(m, pd, dst, zmsg)
        has_next = i < NLAYERS - 1
        j = min(i + 1, NLAYERS - 1)
        outs = _tc_node(i == 0, has_next, h, mp, pp, pos16, deginv,
                        upd_W1[i, :EMB, :], upd_W1[i, EMB:, :],
                        row(upd_b1[i]), row(upd_g1[i]), row(upd_be1[i]),
                        f(upd_W2[i]), row(upd_b2[i]), row(upd_g2[i]),
                        row(upd_be2[i]),
                        msg_W1[j, :EMB, :], msg_W1[j, EMB:2 * EMB, :])
        if has_next:
            h, pos16, t1, t2 = outs[:4]
            if i == 0:
                deginv = outs[4]
        else:
            h = outs[0]

    gp = _sc_pool(h, batch_i, zmsg)
    return _tc_pred(gp, f(pred_W1), row(pred_b1), row(pred_W2[:, 0]),
                    one(pred_b2))
